# baseline (speedup 1.0000x reference)
import functools

import jax
import jax.numpy as jnp
from jax import lax
from jax.experimental import pallas as pl
from jax.experimental.pallas import tpu as pltpu
from jax.experimental.pallas import tpu_sc as plsc

N = 10000
D = 128
E = 320000

NC = 2
NS = 16
NW = NC * NS

NPAD = 10240
B = 128
EPW = 10112
EPAD = EPW * NW
NB = EPW // B
RPT = NPAD // NS

_mesh = plsc.VectorSubcoreMesh(core_axis_name="c", subcore_axis_name="s")


@functools.partial(
    pl.kernel,
    out_type=jax.ShapeDtypeStruct((NC, NPAD), jnp.float32),
    mesh=_mesh,
    scratch_types=[
        pltpu.VMEM((B,), jnp.int32),
        pltpu.VMEM((B,), jnp.int32),
        pltpu.VMEM((B,), jnp.float32),
        pltpu.SemaphoreType.DMA,
        pltpu.SemaphoreType.DMA,
        pltpu.VMEM_SHARED((NPAD,), jnp.float32),
    ],
)
def _deg_kernel(dst_hbm, zeros1_hbm, parts_hbm, idx0_v, idx1_v, ones_v,
                semi0, semi1, acc_sh):
    c = lax.axis_index("c")
    s = lax.axis_index("s")
    w = s * NC + c

    for j in range(B // 16):
        ones_v[pl.ds(j * 16, 16)] = jnp.ones((16,), jnp.float32)

    pltpu.sync_copy(zeros1_hbm.at[pl.ds(s * RPT, RPT)],
                    acc_sh.at[pl.ds(s * RPT, RPT)])
    plsc.subcore_barrier()

    def load(b, idx, sem):
        return pltpu.make_async_copy(
            dst_hbm.at[pl.ds(w * EPW + b * B, B)], idx, sem)

    load(0, idx0_v, semi0).start()

    def step(i, carry):
        b0 = 2 * i
        b1 = 2 * i + 1
        load(b1, idx1_v, semi1).start()
        load(b0, idx0_v, semi0).wait()
        pltpu.sync_copy(ones_v, acc_sh.at[idx0_v], add=True)

        @pl.when(i < NB // 2 - 1)
        def _():
            load(b1 + 1, idx0_v, semi0).start()

        load(b1, idx1_v, semi1).wait()
        pltpu.sync_copy(ones_v, acc_sh.at[idx1_v], add=True)
        return carry

    lax.fori_loop(0, NB // 2, step, 0)

    if NB % 2 == 1:
        load(NB - 1, idx0_v, semi0).start()
        load(NB - 1, idx0_v, semi0).wait()
        pltpu.sync_copy(ones_v, acc_sh.at[idx0_v], add=True)

    plsc.subcore_barrier()
    pltpu.sync_copy(acc_sh.at[pl.ds(s * RPT, RPT)],
                    parts_hbm.at[c, pl.ds(s * RPT, RPT)])


@functools.partial(
    pl.kernel,
    out_type=jax.ShapeDtypeStruct((NC, NPAD, D), jnp.float32),
    mesh=_mesh,
    scratch_types=[
        pltpu.VMEM((B,), jnp.int32),
        pltpu.VMEM((B,), jnp.int32),
        pltpu.VMEM((B,), jnp.int32),
        pltpu.VMEM((B,), jnp.int32),
        pltpu.VMEM((B, D), jnp.float32),
        pltpu.VMEM((B, D), jnp.float32),
        pltpu.SemaphoreType.DMA,
        pltpu.SemaphoreType.DMA,
        pltpu.VMEM_SHARED((NPAD, D), jnp.float32),
    ],
)
def _agg_kernel(hp_hbm, src_hbm, dst_hbm, zeros2_hbm, parts_hbm,
                idxs0_v, idxs1_v, idxd0_v, idxd1_v,
                rows0_v, rows1_v, sem0, sem1, acc_sh):
    c = lax.axis_index("c")
    s = lax.axis_index("s")
    w = s * NC + c

    pltpu.sync_copy(zeros2_hbm.at[pl.ds(s * RPT, RPT)],
                    acc_sh.at[pl.ds(s * RPT, RPT)])
    plsc.subcore_barrier()

    def load_idx(b, idxs, idxd):
        base = w * EPW + b * B
        pltpu.sync_copy(src_hbm.at[pl.ds(base, B)], idxs)
        pltpu.sync_copy(dst_hbm.at[pl.ds(base, B)], idxd)

    def gather0():
        return pltpu.make_async_copy(hp_hbm.at[idxs0_v], rows0_v, sem0)

    def gather1():
        return pltpu.make_async_copy(hp_hbm.at[idxs1_v], rows1_v, sem1)

    load_idx(0, idxs0_v, idxd0_v)
    gather0().start()

    def step(i, carry):
        b1 = 2 * i + 1
        load_idx(b1, idxs1_v, idxd1_v)
        gather1().start()
        gather0().wait()
        pltpu.sync_copy(rows0_v, acc_sh.at[idxd0_v], add=True)

        @pl.when(i < NB // 2 - 1)
        def _():
            load_idx(b1 + 1, idxs0_v, idxd0_v)
            gather0().start()

        gather1().wait()
        pltpu.sync_copy(rows1_v, acc_sh.at[idxd1_v], add=True)
        return carry

    lax.fori_loop(0, NB // 2, step, 0)

    if NB % 2 == 1:
        load_idx(NB - 1, idxs0_v, idxd0_v)
        gather0().start()
        gather0().wait()
        pltpu.sync_copy(rows0_v, acc_sh.at[idxd0_v], add=True)

    plsc.subcore_barrier()
    pltpu.sync_copy(acc_sh.at[pl.ds(s * RPT, RPT)],
                    parts_hbm.at[c, pl.ds(s * RPT, RPT)])


_TCR = 512
_TCG = NPAD // _TCR


def _mm_body(x_ref, w_ref, p0_ref, p1_ref, o_ref):
    dinv = lax.rsqrt(1.0 + p0_ref[...] + p1_ref[...])
    o_ref[...] = jnp.dot(x_ref[...], w_ref[...],
                         preferred_element_type=jnp.float32) * dinv


def _mm(x, w, p0, p1):
    return pl.pallas_call(
        _mm_body,
        grid=(_TCG,),
        in_specs=[
            pl.BlockSpec((_TCR, D), lambda i: (i, 0)),
            pl.BlockSpec((D, D), lambda i: (0, 0)),
            pl.BlockSpec((_TCR, 1), lambda i: (i, 0)),
            pl.BlockSpec((_TCR, 1), lambda i: (i, 0)),
        ],
        out_specs=pl.BlockSpec((_TCR, D), lambda i: (i, 0)),
        out_shape=jax.ShapeDtypeStruct((NPAD, D), jnp.float32),
    )(x, w, p0, p1)


def _ln_gelu(t, g, be):
    mu = jnp.mean(t, axis=-1, keepdims=True)
    dev = t - mu
    var = jnp.mean(dev * dev, axis=-1, keepdims=True)
    y = g * dev * lax.rsqrt(var + 1e-5) + be
    return 0.5 * y * (1.0 + lax.erf(y * 0.7071067811865476))


def _combmm_body(a0_ref, a1_ref, hp_ref, p0_ref, p1_ref, b_ref, g_ref, be_ref,
                 w_ref, o_ref):
    dinv = lax.rsqrt(1.0 + p0_ref[...] + p1_ref[...])
    t = (a0_ref[...] + a1_ref[...] + hp_ref[...]) * dinv + b_ref[...]
    y = _ln_gelu(t, g_ref[...], be_ref[...])
    o_ref[...] = jnp.dot(y, w_ref[...],
                         preferred_element_type=jnp.float32) * dinv


def _combmm(a0, a1, hp, p0, p1, b, g, be, w):
    return pl.pallas_call(
        _combmm_body,
        grid=(_TCG,),
        in_specs=[
            pl.BlockSpec((_TCR, D), lambda i: (i, 0)),
            pl.BlockSpec((_TCR, D), lambda i: (i, 0)),
            pl.BlockSpec((_TCR, D), lambda i: (i, 0)),
            pl.BlockSpec((_TCR, 1), lambda i: (i, 0)),
            pl.BlockSpec((_TCR, 1), lambda i: (i, 0)),
            pl.BlockSpec((1, D), lambda i: (0, 0)),
            pl.BlockSpec((1, D), lambda i: (0, 0)),
            pl.BlockSpec((1, D), lambda i: (0, 0)),
            pl.BlockSpec((D, D), lambda i: (0, 0)),
        ],
        out_specs=pl.BlockSpec((_TCR, D), lambda i: (i, 0)),
        out_shape=jax.ShapeDtypeStruct((NPAD, D), jnp.float32),
    )(a0, a1, hp, p0, p1, b, g, be, w)


_FR = 400
_FG = N // _FR


def _comb_body(a0_ref, a1_ref, hp_ref, p0_ref, p1_ref, b_ref, g_ref, be_ref,
               o_ref):
    dinv = lax.rsqrt(1.0 + p0_ref[...] + p1_ref[...])
    t = (a0_ref[...] + a1_ref[...] + hp_ref[...]) * dinv + b_ref[...]
    o_ref[...] = _ln_gelu(t, g_ref[...], be_ref[...])


def _comb(a0, a1, hp, p0, p1, b, g, be):
    return pl.pallas_call(
        _comb_body,
        grid=(_FG,),
        in_specs=[
            pl.BlockSpec((_FR, D), lambda i: (i, 0)),
            pl.BlockSpec((_FR, D), lambda i: (i, 0)),
            pl.BlockSpec((_FR, D), lambda i: (i, 0)),
            pl.BlockSpec((_FR, 1), lambda i: (i, 0)),
            pl.BlockSpec((_FR, 1), lambda i: (i, 0)),
            pl.BlockSpec((1, D), lambda i: (0, 0)),
            pl.BlockSpec((1, D), lambda i: (0, 0)),
            pl.BlockSpec((1, D), lambda i: (0, 0)),
        ],
        out_specs=pl.BlockSpec((_FR, D), lambda i: (i, 0)),
        out_shape=jax.ShapeDtypeStruct((N, D), jnp.float32),
    )(a0, a1, hp, p0, p1, b, g, be)


@jax.jit
def kernel(x, edge_index, W1, b1, g1, be1, W2, b2, g2, be2):
    f32 = jnp.float32
    xp = jnp.pad(x, ((0, NPAD - N), (0, 0)))
    pad = jnp.full((EPAD - E,), NPAD - 1, dtype=jnp.int32)
    srcp = jnp.concatenate([edge_index[0], pad])
    dstp = jnp.concatenate([edge_index[1], pad])
    zeros1 = jnp.zeros((NPAD,), f32)
    zeros2 = jnp.zeros((NPAD, D), f32)

    degp = _deg_kernel(dstp, zeros1)
    p0 = degp[0].reshape(NPAD, 1)
    p1 = degp[1].reshape(NPAD, 1)

    b1r = b1.reshape(1, D)
    g1r = g1.reshape(1, D)
    be1r = be1.reshape(1, D)
    b2r = b2.reshape(1, D)
    g2r = g2.reshape(1, D)
    be2r = be2.reshape(1, D)

    hp1 = _mm(xp, W1, p0, p1)
    agg1 = _agg_kernel(hp1, srcp, dstp, zeros2)
    hp2 = _combmm(agg1[0], agg1[1], hp1, p0, p1, b1r, g1r, be1r, W2)
    agg2 = _agg_kernel(hp2, srcp, dstp, zeros2)
    return _comb(agg2[0], agg2[1], hp2, p0, p1, b2r, g2r, be2r)


# speedup vs baseline: 1.0001x; 1.0001x over previous
import functools

import jax
import jax.numpy as jnp
from jax import lax
from jax.experimental import pallas as pl
from jax.experimental.pallas import tpu as pltpu
from jax.experimental.pallas import tpu_sc as plsc

N = 10000
D = 128
E = 320000

NC = 2
NS = 16
NW = NC * NS

NPAD = 10240
B = 128
EPW = 10112
EPAD = EPW * NW
NB = EPW // B
RPT = NPAD // NS

_mesh = plsc.VectorSubcoreMesh(core_axis_name="c", subcore_axis_name="s")


@functools.partial(
    pl.kernel,
    out_type=jax.ShapeDtypeStruct((NC, NPAD), jnp.float32),
    mesh=_mesh,
    scratch_types=[
        pltpu.VMEM((B,), jnp.int32),
        pltpu.VMEM((B,), jnp.int32),
        pltpu.VMEM((B,), jnp.float32),
        pltpu.SemaphoreType.DMA,
        pltpu.SemaphoreType.DMA,
        pltpu.VMEM_SHARED((NPAD,), jnp.float32),
    ],
)
def _deg_kernel(dst_hbm, zeros1_hbm, parts_hbm, idx0_v, idx1_v, ones_v,
                semi0, semi1, acc_sh):
    c = lax.axis_index("c")
    s = lax.axis_index("s")
    w = s * NC + c

    for j in range(B // 16):
        ones_v[pl.ds(j * 16, 16)] = jnp.ones((16,), jnp.float32)

    pltpu.sync_copy(zeros1_hbm.at[pl.ds(s * RPT, RPT)],
                    acc_sh.at[pl.ds(s * RPT, RPT)])
    plsc.subcore_barrier()

    def load(b, idx, sem):
        return pltpu.make_async_copy(
            dst_hbm.at[pl.ds(w * EPW + b * B, B)], idx, sem)

    load(0, idx0_v, semi0).start()

    def step(i, carry):
        b0 = 2 * i
        b1 = 2 * i + 1
        load(b1, idx1_v, semi1).start()
        load(b0, idx0_v, semi0).wait()
        pltpu.sync_copy(ones_v, acc_sh.at[idx0_v], add=True)

        @pl.when(i < NB // 2 - 1)
        def _():
            load(b1 + 1, idx0_v, semi0).start()

        load(b1, idx1_v, semi1).wait()
        pltpu.sync_copy(ones_v, acc_sh.at[idx1_v], add=True)
        return carry

    lax.fori_loop(0, NB // 2, step, 0)

    if NB % 2 == 1:
        load(NB - 1, idx0_v, semi0).start()
        load(NB - 1, idx0_v, semi0).wait()
        pltpu.sync_copy(ones_v, acc_sh.at[idx0_v], add=True)

    plsc.subcore_barrier()
    pltpu.sync_copy(acc_sh.at[pl.ds(s * RPT, RPT)],
                    parts_hbm.at[c, pl.ds(s * RPT, RPT)])


@functools.partial(
    pl.kernel,
    out_type=jax.ShapeDtypeStruct((NC, NPAD, D), jnp.float32),
    mesh=_mesh,
    scratch_types=[
        pltpu.VMEM((B,), jnp.int32),
        pltpu.VMEM((B,), jnp.int32),
        pltpu.VMEM((B,), jnp.int32),
        pltpu.VMEM((B,), jnp.int32),
        pltpu.VMEM((B, D), jnp.float32),
        pltpu.VMEM((B, D), jnp.float32),
        pltpu.SemaphoreType.DMA,
        pltpu.SemaphoreType.DMA,
        pltpu.SemaphoreType.DMA,
        pltpu.SemaphoreType.DMA,
        pltpu.VMEM_SHARED((NPAD, D), jnp.float32),
    ],
)
def _agg_kernel(hp_hbm, src_hbm, dst_hbm, zeros2_hbm, parts_hbm,
                idxs0_v, idxs1_v, idxd0_v, idxd1_v,
                rows0_v, rows1_v, sem0, sem1, sems0, sems1, acc_sh):
    c = lax.axis_index("c")
    s = lax.axis_index("s")
    w = s * NC + c

    pltpu.sync_copy(zeros2_hbm.at[pl.ds(s * RPT, RPT)],
                    acc_sh.at[pl.ds(s * RPT, RPT)])
    plsc.subcore_barrier()

    def load_idx(b, idxs, idxd):
        base = w * EPW + b * B
        pltpu.sync_copy(src_hbm.at[pl.ds(base, B)], idxs)
        pltpu.sync_copy(dst_hbm.at[pl.ds(base, B)], idxd)

    def gather0():
        return pltpu.make_async_copy(hp_hbm.at[idxs0_v], rows0_v, sem0)

    def gather1():
        return pltpu.make_async_copy(hp_hbm.at[idxs1_v], rows1_v, sem1)

    def scat0():
        return pltpu.make_async_copy(rows0_v, acc_sh.at[idxd0_v], sems0)

    def scat1():
        return pltpu.make_async_copy(rows1_v, acc_sh.at[idxd1_v], sems1)

    load_idx(0, idxs0_v, idxd0_v)
    gather0().start()

    def step(i, carry):
        b1 = 2 * i + 1

        @pl.when(i > 0)
        def _():
            scat1().wait()

        load_idx(b1, idxs1_v, idxd1_v)
        gather1().start()
        gather0().wait()
        pltpu.async_copy(rows0_v, acc_sh.at[idxd0_v], sems0, add=True)

        @pl.when(i < NB // 2 - 1)
        def _():
            scat0().wait()
            load_idx(b1 + 1, idxs0_v, idxd0_v)
            gather0().start()

        gather1().wait()
        pltpu.async_copy(rows1_v, acc_sh.at[idxd1_v], sems1, add=True)
        return carry

    lax.fori_loop(0, NB // 2, step, 0)
    scat1().wait()

    if NB % 2 == 1:
        scat0().wait()
        load_idx(NB - 1, idxs0_v, idxd0_v)
        gather0().start()
        gather0().wait()
        pltpu.sync_copy(rows0_v, acc_sh.at[idxd0_v], add=True)
    else:
        scat0().wait()

    plsc.subcore_barrier()
    pltpu.sync_copy(acc_sh.at[pl.ds(s * RPT, RPT)],
                    parts_hbm.at[c, pl.ds(s * RPT, RPT)])


_TCR = 512
_TCG = NPAD // _TCR


def _mm_body(x_ref, w_ref, p0_ref, p1_ref, o_ref):
    dinv = lax.rsqrt(1.0 + p0_ref[...] + p1_ref[...])
    o_ref[...] = jnp.dot(x_ref[...], w_ref[...],
                         preferred_element_type=jnp.float32) * dinv


def _mm(x, w, p0, p1):
    return pl.pallas_call(
        _mm_body,
        grid=(_TCG,),
        in_specs=[
            pl.BlockSpec((_TCR, D), lambda i: (i, 0)),
            pl.BlockSpec((D, D), lambda i: (0, 0)),
            pl.BlockSpec((_TCR, 1), lambda i: (i, 0)),
            pl.BlockSpec((_TCR, 1), lambda i: (i, 0)),
        ],
        out_specs=pl.BlockSpec((_TCR, D), lambda i: (i, 0)),
        out_shape=jax.ShapeDtypeStruct((NPAD, D), jnp.float32),
    )(x, w, p0, p1)


def _ln_gelu(t, g, be):
    mu = jnp.mean(t, axis=-1, keepdims=True)
    dev = t - mu
    var = jnp.mean(dev * dev, axis=-1, keepdims=True)
    y = g * dev * lax.rsqrt(var + 1e-5) + be
    return 0.5 * y * (1.0 + lax.erf(y * 0.7071067811865476))


def _combmm_body(a0_ref, a1_ref, hp_ref, p0_ref, p1_ref, b_ref, g_ref, be_ref,
                 w_ref, o_ref):
    dinv = lax.rsqrt(1.0 + p0_ref[...] + p1_ref[...])
    t = (a0_ref[...] + a1_ref[...] + hp_ref[...]) * dinv + b_ref[...]
    y = _ln_gelu(t, g_ref[...], be_ref[...])
    o_ref[...] = jnp.dot(y, w_ref[...],
                         preferred_element_type=jnp.float32) * dinv


def _combmm(a0, a1, hp, p0, p1, b, g, be, w):
    return pl.pallas_call(
        _combmm_body,
        grid=(_TCG,),
        in_specs=[
            pl.BlockSpec((_TCR, D), lambda i: (i, 0)),
            pl.BlockSpec((_TCR, D), lambda i: (i, 0)),
            pl.BlockSpec((_TCR, D), lambda i: (i, 0)),
            pl.BlockSpec((_TCR, 1), lambda i: (i, 0)),
            pl.BlockSpec((_TCR, 1), lambda i: (i, 0)),
            pl.BlockSpec((1, D), lambda i: (0, 0)),
            pl.BlockSpec((1, D), lambda i: (0, 0)),
            pl.BlockSpec((1, D), lambda i: (0, 0)),
            pl.BlockSpec((D, D), lambda i: (0, 0)),
        ],
        out_specs=pl.BlockSpec((_TCR, D), lambda i: (i, 0)),
        out_shape=jax.ShapeDtypeStruct((NPAD, D), jnp.float32),
    )(a0, a1, hp, p0, p1, b, g, be, w)


_FR = 400
_FG = N // _FR


def _comb_body(a0_ref, a1_ref, hp_ref, p0_ref, p1_ref, b_ref, g_ref, be_ref,
               o_ref):
    dinv = lax.rsqrt(1.0 + p0_ref[...] + p1_ref[...])
    t = (a0_ref[...] + a1_ref[...] + hp_ref[...]) * dinv + b_ref[...]
    o_ref[...] = _ln_gelu(t, g_ref[...], be_ref[...])


def _comb(a0, a1, hp, p0, p1, b, g, be):
    return pl.pallas_call(
        _comb_body,
        grid=(_FG,),
        in_specs=[
            pl.BlockSpec((_FR, D), lambda i: (i, 0)),
            pl.BlockSpec((_FR, D), lambda i: (i, 0)),
            pl.BlockSpec((_FR, D), lambda i: (i, 0)),
            pl.BlockSpec((_FR, 1), lambda i: (i, 0)),
            pl.BlockSpec((_FR, 1), lambda i: (i, 0)),
            pl.BlockSpec((1, D), lambda i: (0, 0)),
            pl.BlockSpec((1, D), lambda i: (0, 0)),
            pl.BlockSpec((1, D), lambda i: (0, 0)),
        ],
        out_specs=pl.BlockSpec((_FR, D), lambda i: (i, 0)),
        out_shape=jax.ShapeDtypeStruct((N, D), jnp.float32),
    )(a0, a1, hp, p0, p1, b, g, be)


@jax.jit
def kernel(x, edge_index, W1, b1, g1, be1, W2, b2, g2, be2):
    f32 = jnp.float32
    xp = jnp.pad(x, ((0, NPAD - N), (0, 0)))
    pad = jnp.full((EPAD - E,), NPAD - 1, dtype=jnp.int32)
    srcp = jnp.concatenate([edge_index[0], pad])
    dstp = jnp.concatenate([edge_index[1], pad])
    zeros1 = jnp.zeros((NPAD,), f32)
    zeros2 = jnp.zeros((NPAD, D), f32)

    degp = _deg_kernel(dstp, zeros1)
    p0 = degp[0].reshape(NPAD, 1)
    p1 = degp[1].reshape(NPAD, 1)

    b1r = b1.reshape(1, D)
    g1r = g1.reshape(1, D)
    be1r = be1.reshape(1, D)
    b2r = b2.reshape(1, D)
    g2r = g2.reshape(1, D)
    be2r = be2.reshape(1, D)

    hp1 = _mm(xp, W1, p0, p1)
    agg1 = _agg_kernel(hp1, srcp, dstp, zeros2)
    hp2 = _combmm(agg1[0], agg1[1], hp1, p0, p1, b1r, g1r, be1r, W2)
    agg2 = _agg_kernel(hp2, srcp, dstp, zeros2)
    return _comb(agg2[0], agg2[1], hp2, p0, p1, b2r, g2r, be2r)


# speedup vs baseline: 1.0504x; 1.0502x over previous
import functools

import jax
import jax.numpy as jnp
from jax import lax
from jax.experimental import pallas as pl
from jax.experimental.pallas import tpu as pltpu
from jax.experimental.pallas import tpu_sc as plsc

N = 10000
D = 128
E = 320000

NC = 2
NS = 16
NW = NC * NS

NPAD = 10240
B = 128
EPW = 10112
EPAD = EPW * NW
NB = EPW // B
RPT = NPAD // NS

_mesh = plsc.VectorSubcoreMesh(core_axis_name="c", subcore_axis_name="s")


@functools.partial(
    pl.kernel,
    out_type=jax.ShapeDtypeStruct((NC, NPAD), jnp.float32),
    mesh=_mesh,
    scratch_types=[
        pltpu.VMEM((B,), jnp.int32),
        pltpu.VMEM((B,), jnp.int32),
        pltpu.VMEM((B,), jnp.float32),
        pltpu.SemaphoreType.DMA,
        pltpu.SemaphoreType.DMA,
        pltpu.VMEM_SHARED((NPAD,), jnp.float32),
    ],
)
def _deg_kernel(dst_hbm, zeros1_hbm, parts_hbm, idx0_v, idx1_v, ones_v,
                semi0, semi1, acc_sh):
    c = lax.axis_index("c")
    s = lax.axis_index("s")
    w = s * NC + c

    for j in range(B // 16):
        ones_v[pl.ds(j * 16, 16)] = jnp.ones((16,), jnp.float32)

    pltpu.sync_copy(zeros1_hbm.at[pl.ds(s * RPT, RPT)],
                    acc_sh.at[pl.ds(s * RPT, RPT)])
    plsc.subcore_barrier()

    def load(b, idx, sem):
        return pltpu.make_async_copy(
            dst_hbm.at[pl.ds(w * EPW + b * B, B)], idx, sem)

    load(0, idx0_v, semi0).start()

    def step(i, carry):
        b0 = 2 * i
        b1 = 2 * i + 1
        load(b1, idx1_v, semi1).start()
        load(b0, idx0_v, semi0).wait()
        pltpu.sync_copy(ones_v, acc_sh.at[idx0_v], add=True)

        @pl.when(i < NB // 2 - 1)
        def _():
            load(b1 + 1, idx0_v, semi0).start()

        load(b1, idx1_v, semi1).wait()
        pltpu.sync_copy(ones_v, acc_sh.at[idx1_v], add=True)
        return carry

    lax.fori_loop(0, NB // 2, step, 0)

    if NB % 2 == 1:
        load(NB - 1, idx0_v, semi0).start()
        load(NB - 1, idx0_v, semi0).wait()
        pltpu.sync_copy(ones_v, acc_sh.at[idx0_v], add=True)

    plsc.subcore_barrier()
    pltpu.sync_copy(acc_sh.at[pl.ds(s * RPT, RPT)],
                    parts_hbm.at[c, pl.ds(s * RPT, RPT)])


@functools.partial(
    pl.kernel,
    out_type=jax.ShapeDtypeStruct((NC, NPAD, D), jnp.float32),
    mesh=_mesh,
    scratch_types=[
        pltpu.VMEM((B,), jnp.int32),
        pltpu.VMEM((B,), jnp.int32),
        pltpu.VMEM((B,), jnp.int32),
        pltpu.VMEM((B,), jnp.int32),
        pltpu.VMEM((B, D), jnp.float32),
        pltpu.VMEM((B, D), jnp.float32),
        pltpu.SemaphoreType.DMA,
        pltpu.SemaphoreType.DMA,
        pltpu.VMEM_SHARED((NPAD, D), jnp.float32),
    ],
)
def _agg_kernel(hp_hbm, src_hbm, dst_hbm, zeros2_hbm, parts_hbm,
                idxs0_v, idxs1_v, idxd0_v, idxd1_v,
                rows0_v, rows1_v, sem0, sem1, acc_sh):
    c = lax.axis_index("c")
    s = lax.axis_index("s")
    w = s * NC + c

    pltpu.sync_copy(zeros2_hbm.at[pl.ds(s * RPT, RPT)],
                    acc_sh.at[pl.ds(s * RPT, RPT)])
    plsc.subcore_barrier()

    def load_idx(b, idxs, idxd):
        base = w * EPW + b * B
        pltpu.sync_copy(src_hbm.at[pl.ds(base, B)], idxs)
        pltpu.sync_copy(dst_hbm.at[pl.ds(base, B)], idxd)

    def gather0():
        return pltpu.make_async_copy(hp_hbm.at[idxs0_v], rows0_v, sem0)

    def gather1():
        return pltpu.make_async_copy(hp_hbm.at[idxs1_v], rows1_v, sem1)

    load_idx(0, idxs0_v, idxd0_v)
    gather0().start()

    def step(i, carry):
        b1 = 2 * i + 1
        load_idx(b1, idxs1_v, idxd1_v)
        gather1().start()
        gather0().wait()
        pltpu.sync_copy(rows0_v, acc_sh.at[idxd0_v], add=True)

        @pl.when(i < NB // 2 - 1)
        def _():
            load_idx(b1 + 1, idxs0_v, idxd0_v)
            gather0().start()

        gather1().wait()
        pltpu.sync_copy(rows1_v, acc_sh.at[idxd1_v], add=True)
        return carry

    lax.fori_loop(0, NB // 2, step, 0)

    if NB % 2 == 1:
        load_idx(NB - 1, idxs0_v, idxd0_v)
        gather0().start()
        gather0().wait()
        pltpu.sync_copy(rows0_v, acc_sh.at[idxd0_v], add=True)

    plsc.subcore_barrier()
    pltpu.sync_copy(acc_sh.at[pl.ds(s * RPT, RPT)],
                    parts_hbm.at[c, pl.ds(s * RPT, RPT)])


_TCR = 512
_TCG = NPAD // _TCR


def _mm_body(x_ref, w_ref, p0_ref, p1_ref, o_ref):
    dinv = lax.rsqrt(1.0 + p0_ref[...] + p1_ref[...])
    o_ref[...] = jnp.dot(x_ref[...], w_ref[...],
                         preferred_element_type=jnp.float32) * dinv


def _mm(x, w, p0, p1):
    return pl.pallas_call(
        _mm_body,
        grid=(_TCG,),
        in_specs=[
            pl.BlockSpec((_TCR, D), lambda i: (i, 0)),
            pl.BlockSpec((D, D), lambda i: (0, 0)),
            pl.BlockSpec((_TCR, 1), lambda i: (i, 0)),
            pl.BlockSpec((_TCR, 1), lambda i: (i, 0)),
        ],
        out_specs=pl.BlockSpec((_TCR, D), lambda i: (i, 0)),
        out_shape=jax.ShapeDtypeStruct((NPAD, D), jnp.float32),
    )(x, w, p0, p1)


def _comb_body(a0_ref, a1_ref, hp_ref, p0_ref, p1_ref, b_ref, g_ref, be_ref,
               o_ref):
    dinv = lax.rsqrt(1.0 + p0_ref[...] + p1_ref[...])
    t = (a0_ref[...] + a1_ref[...] + hp_ref[...]) * dinv + b_ref[...]
    mu = jnp.mean(t, axis=-1, keepdims=True)
    dev = t - mu
    var = jnp.mean(dev * dev, axis=-1, keepdims=True)
    y = g_ref[...] * dev * lax.rsqrt(var + 1e-5) + be_ref[...]
    o_ref[...] = 0.5 * y * (1.0 + lax.erf(y * 0.7071067811865476))


def _comb(a0, a1, hp, p0, p1, b, g, be):
    return pl.pallas_call(
        _comb_body,
        grid=(_TCG,),
        in_specs=[
            pl.BlockSpec((_TCR, D), lambda i: (i, 0)),
            pl.BlockSpec((_TCR, D), lambda i: (i, 0)),
            pl.BlockSpec((_TCR, D), lambda i: (i, 0)),
            pl.BlockSpec((_TCR, 1), lambda i: (i, 0)),
            pl.BlockSpec((_TCR, 1), lambda i: (i, 0)),
            pl.BlockSpec((1, D), lambda i: (0, 0)),
            pl.BlockSpec((1, D), lambda i: (0, 0)),
            pl.BlockSpec((1, D), lambda i: (0, 0)),
        ],
        out_specs=pl.BlockSpec((_TCR, D), lambda i: (i, 0)),
        out_shape=jax.ShapeDtypeStruct((NPAD, D), jnp.float32),
    )(a0, a1, hp, p0, p1, b, g, be)


@jax.jit
def kernel(x, edge_index, W1, b1, g1, be1, W2, b2, g2, be2):
    f32 = jnp.float32
    xp = jnp.pad(x, ((0, NPAD - N), (0, 0)))
    pad = jnp.full((EPAD - E,), NPAD - 1, dtype=jnp.int32)
    srcp = jnp.concatenate([edge_index[0], pad])
    dstp = jnp.concatenate([edge_index[1], pad])
    zeros1 = jnp.zeros((NPAD,), f32)
    zeros2 = jnp.zeros((NPAD, D), f32)

    degp = _deg_kernel(dstp, zeros1)
    p0 = degp[0].reshape(NPAD, 1)
    p1 = degp[1].reshape(NPAD, 1)

    b1r = b1.reshape(1, D)
    g1r = g1.reshape(1, D)
    be1r = be1.reshape(1, D)
    b2r = b2.reshape(1, D)
    g2r = g2.reshape(1, D)
    be2r = be2.reshape(1, D)

    hp1 = _mm(xp, W1, p0, p1)
    agg1 = _agg_kernel(hp1, srcp, dstp, zeros2)
    x2 = _comb(agg1[0], agg1[1], hp1, p0, p1, b1r, g1r, be1r)

    hp2 = _mm(x2, W2, p0, p1)
    agg2 = _agg_kernel(hp2, srcp, dstp, zeros2)
    out = _comb(agg2[0], agg2[1], hp2, p0, p1, b2r, g2r, be2r)

    return out[:N]


# speedup vs baseline: 1.1064x; 1.0533x over previous
import functools

import jax
import jax.numpy as jnp
from jax import lax
from jax.experimental import pallas as pl
from jax.experimental.pallas import tpu as pltpu
from jax.experimental.pallas import tpu_sc as plsc

N = 10000
D = 128
E = 320000

NC = 2
NS = 16
NW = NC * NS

NPAD = 10240
B = 128
EPW = 10112
EPAD = EPW * NW
NB = EPW // B
RPT = NPAD // NS

_mesh = plsc.VectorSubcoreMesh(core_axis_name="c", subcore_axis_name="s")


@functools.partial(
    pl.kernel,
    out_type=jax.ShapeDtypeStruct((NC, NPAD), jnp.float32),
    mesh=_mesh,
    scratch_types=[
        pltpu.VMEM((B,), jnp.int32),
        pltpu.VMEM((B,), jnp.int32),
        pltpu.VMEM((B,), jnp.float32),
        pltpu.SemaphoreType.DMA,
        pltpu.SemaphoreType.DMA,
        pltpu.VMEM_SHARED((NPAD,), jnp.float32),
    ],
)
def _deg_kernel(dst_hbm, zeros1_hbm, parts_hbm, idx0_v, idx1_v, ones_v,
                semi0, semi1, acc_sh):
    c = lax.axis_index("c")
    s = lax.axis_index("s")
    w = s * NC + c

    for j in range(B // 16):
        ones_v[pl.ds(j * 16, 16)] = jnp.ones((16,), jnp.float32)

    pltpu.sync_copy(zeros1_hbm.at[pl.ds(s * RPT, RPT)],
                    acc_sh.at[pl.ds(s * RPT, RPT)])
    plsc.subcore_barrier()

    def load(b, idx, sem):
        return pltpu.make_async_copy(
            dst_hbm.at[pl.ds(w * EPW + b * B, B)], idx, sem)

    load(0, idx0_v, semi0).start()

    def step(i, carry):
        b0 = 2 * i
        b1 = 2 * i + 1
        load(b1, idx1_v, semi1).start()
        load(b0, idx0_v, semi0).wait()
        pltpu.sync_copy(ones_v, acc_sh.at[idx0_v], add=True)

        @pl.when(i < NB // 2 - 1)
        def _():
            load(b1 + 1, idx0_v, semi0).start()

        load(b1, idx1_v, semi1).wait()
        pltpu.sync_copy(ones_v, acc_sh.at[idx1_v], add=True)
        return carry

    lax.fori_loop(0, NB // 2, step, 0)

    if NB % 2 == 1:
        load(NB - 1, idx0_v, semi0).start()
        load(NB - 1, idx0_v, semi0).wait()
        pltpu.sync_copy(ones_v, acc_sh.at[idx0_v], add=True)

    plsc.subcore_barrier()
    pltpu.sync_copy(acc_sh.at[pl.ds(s * RPT, RPT)],
                    parts_hbm.at[c, pl.ds(s * RPT, RPT)])


@functools.partial(
    pl.kernel,
    out_type=jax.ShapeDtypeStruct((NC, NPAD, D), jnp.float32),
    mesh=_mesh,
    scratch_types=[
        pltpu.VMEM((B,), jnp.int32),
        pltpu.VMEM((B,), jnp.int32),
        pltpu.VMEM((B,), jnp.int32),
        pltpu.VMEM((B,), jnp.int32),
        pltpu.VMEM((B, D), jnp.float32),
        pltpu.VMEM((B, D), jnp.float32),
        pltpu.SemaphoreType.DMA,
        pltpu.SemaphoreType.DMA,
        pltpu.SemaphoreType.DMA,
        pltpu.SemaphoreType.DMA,
        pltpu.VMEM_SHARED((NPAD, D), jnp.float32),
    ],
)
def _agg_kernel(hp_hbm, src_hbm, dst_hbm, zeros2_hbm, parts_hbm,
                idxs0_v, idxs1_v, idxd0_v, idxd1_v,
                rows0_v, rows1_v, sem0, sem1, semi0, semi1, acc_sh):
    c = lax.axis_index("c")
    s = lax.axis_index("s")
    w = s * NC + c

    pltpu.sync_copy(zeros2_hbm.at[pl.ds(s * RPT, RPT)],
                    acc_sh.at[pl.ds(s * RPT, RPT)])
    plsc.subcore_barrier()

    def load_src(b, idxs, semi):
        return pltpu.make_async_copy(
            src_hbm.at[pl.ds(w * EPW + b * B, B)], idxs, semi)

    def load_dst(b, idxd):
        pltpu.sync_copy(dst_hbm.at[pl.ds(w * EPW + b * B, B)], idxd)

    def gather0():
        return pltpu.make_async_copy(hp_hbm.at[idxs0_v], rows0_v, sem0)

    def gather1():
        return pltpu.make_async_copy(hp_hbm.at[idxs1_v], rows1_v, sem1)

    load_src(0, idxs0_v, semi0).start()
    load_src(0, idxs0_v, semi0).wait()
    load_dst(0, idxd0_v)
    gather0().start()
    load_src(1, idxs1_v, semi1).start()

    def step(i, carry):
        b1 = 2 * i + 1
        load_src(b1, idxs1_v, semi1).wait()
        load_dst(b1, idxd1_v)
        gather1().start()
        gather0().wait()

        @pl.when(i < NB // 2 - 1)
        def _():
            load_src(b1 + 1, idxs0_v, semi0).start()

        pltpu.sync_copy(rows0_v, acc_sh.at[idxd0_v], add=True)

        @pl.when(i < NB // 2 - 1)
        def _():
            load_src(b1 + 1, idxs0_v, semi0).wait()
            load_dst(b1 + 1, idxd0_v)
            gather0().start()

        gather1().wait()

        @pl.when(i < NB // 2 - 1)
        def _():
            load_src(b1 + 2, idxs1_v, semi1).start()

        pltpu.sync_copy(rows1_v, acc_sh.at[idxd1_v], add=True)
        return carry

    lax.fori_loop(0, NB // 2, step, 0)

    if NB % 2 == 1:
        load_src(NB - 1, idxs0_v, semi0).start()
        load_src(NB - 1, idxs0_v, semi0).wait()
        load_dst(NB - 1, idxd0_v)
        gather0().start()
        gather0().wait()
        pltpu.sync_copy(rows0_v, acc_sh.at[idxd0_v], add=True)

    plsc.subcore_barrier()
    pltpu.sync_copy(acc_sh.at[pl.ds(s * RPT, RPT)],
                    parts_hbm.at[c, pl.ds(s * RPT, RPT)])


_TCR = 512
_TCG = NPAD // _TCR


def _mm_body(x_ref, w_ref, p0_ref, p1_ref, o_ref):
    dinv = lax.rsqrt(1.0 + p0_ref[...] + p1_ref[...])
    o_ref[...] = jnp.dot(x_ref[...], w_ref[...],
                         preferred_element_type=jnp.float32) * dinv


def _mm(x, w, p0, p1):
    return pl.pallas_call(
        _mm_body,
        grid=(_TCG,),
        in_specs=[
            pl.BlockSpec((_TCR, D), lambda i: (i, 0)),
            pl.BlockSpec((D, D), lambda i: (0, 0)),
            pl.BlockSpec((_TCR, 1), lambda i: (i, 0)),
            pl.BlockSpec((_TCR, 1), lambda i: (i, 0)),
        ],
        out_specs=pl.BlockSpec((_TCR, D), lambda i: (i, 0)),
        out_shape=jax.ShapeDtypeStruct((NPAD, D), jnp.float32),
    )(x, w, p0, p1)


def _comb_body(a0_ref, a1_ref, hp_ref, p0_ref, p1_ref, b_ref, g_ref, be_ref,
               o_ref):
    dinv = lax.rsqrt(1.0 + p0_ref[...] + p1_ref[...])
    t = (a0_ref[...] + a1_ref[...] + hp_ref[...]) * dinv + b_ref[...]
    mu = jnp.mean(t, axis=-1, keepdims=True)
    dev = t - mu
    var = jnp.mean(dev * dev, axis=-1, keepdims=True)
    y = g_ref[...] * dev * lax.rsqrt(var + 1e-5) + be_ref[...]
    o_ref[...] = 0.5 * y * (1.0 + lax.erf(y * 0.7071067811865476))


def _comb(a0, a1, hp, p0, p1, b, g, be):
    return pl.pallas_call(
        _comb_body,
        grid=(_TCG,),
        in_specs=[
            pl.BlockSpec((_TCR, D), lambda i: (i, 0)),
            pl.BlockSpec((_TCR, D), lambda i: (i, 0)),
            pl.BlockSpec((_TCR, D), lambda i: (i, 0)),
            pl.BlockSpec((_TCR, 1), lambda i: (i, 0)),
            pl.BlockSpec((_TCR, 1), lambda i: (i, 0)),
            pl.BlockSpec((1, D), lambda i: (0, 0)),
            pl.BlockSpec((1, D), lambda i: (0, 0)),
            pl.BlockSpec((1, D), lambda i: (0, 0)),
        ],
        out_specs=pl.BlockSpec((_TCR, D), lambda i: (i, 0)),
        out_shape=jax.ShapeDtypeStruct((NPAD, D), jnp.float32),
    )(a0, a1, hp, p0, p1, b, g, be)


@jax.jit
def kernel(x, edge_index, W1, b1, g1, be1, W2, b2, g2, be2):
    f32 = jnp.float32
    xp = jnp.pad(x, ((0, NPAD - N), (0, 0)))
    pad = jnp.full((EPAD - E,), NPAD - 1, dtype=jnp.int32)
    srcp = jnp.concatenate([edge_index[0], pad])
    dstp = jnp.concatenate([edge_index[1], pad])
    zeros1 = jnp.zeros((NPAD,), f32)
    zeros2 = jnp.zeros((NPAD, D), f32)

    degp = _deg_kernel(dstp, zeros1)
    p0 = degp[0].reshape(NPAD, 1)
    p1 = degp[1].reshape(NPAD, 1)

    b1r = b1.reshape(1, D)
    g1r = g1.reshape(1, D)
    be1r = be1.reshape(1, D)
    b2r = b2.reshape(1, D)
    g2r = g2.reshape(1, D)
    be2r = be2.reshape(1, D)

    hp1 = _mm(xp, W1, p0, p1)
    agg1 = _agg_kernel(hp1, srcp, dstp, zeros2)
    x2 = _comb(agg1[0], agg1[1], hp1, p0, p1, b1r, g1r, be1r)

    hp2 = _mm(x2, W2, p0, p1)
    agg2 = _agg_kernel(hp2, srcp, dstp, zeros2)
    out = _comb(agg2[0], agg2[1], hp2, p0, p1, b2r, g2r, be2r)

    return out[:N]
